# speedup vs baseline: 1.7658x; 1.7658x over previous
import functools

import jax
import jax.numpy as jnp
from jax import lax
from jax.experimental import pallas as pl
from jax.experimental.pallas import tpu as pltpu
from jax.experimental.pallas import tpu_sc as plsc

D = 128
BATCH = 4096
K = 10
NS = 5
NCH = K + NS
NW = 32
BPW = BATCH // NW


def _sc_gather_body(idx_hbm, emb_hbm, out_hbm, idx_v, rows_v, sem):
    c = lax.axis_index("c")
    s = lax.axis_index("s")
    wid = s * 2 + c
    base = wid * BPW
    pltpu.sync_copy(idx_hbm.at[:, pl.ds(base, BPW)], idx_v)
    for k in range(NCH):
        pltpu.async_copy(emb_hbm.at[idx_v.at[k]], rows_v, sem).wait()
        pltpu.sync_copy(rows_v, out_hbm.at[k, pl.ds(base, BPW)])


@jax.jit
def _sc_gather(idx_all, embedding):
    mesh = plsc.VectorSubcoreMesh(core_axis_name="c", subcore_axis_name="s")
    return pl.kernel(
        _sc_gather_body,
        out_type=jax.ShapeDtypeStruct((NCH, BATCH, D), jnp.float32),
        mesh=mesh,
        scratch_types=[
            pltpu.VMEM((NCH, BPW), jnp.int32),
            pltpu.VMEM((BPW, D), jnp.float32),
            pltpu.SemaphoreType.DMA,
        ],
    )(idx_all, embedding)


_BB = 256
_NBLK = BATCH // _BB


def _tc_loss_body(g_ref, out_ref):
    i = pl.program_id(0)
    x0 = g_ref[0:1]
    rest = g_ref[1:]
    dots = jnp.sum(x0 * rest, axis=-1)
    pos = dots[: K - 1]
    neg = dots[K - 1:]
    pos_loss = -jnp.log(jax.nn.sigmoid(pos) + 1e-08)
    neg_loss = -jnp.log(1.0 - jax.nn.sigmoid(neg) + 1e-08)
    part = (jnp.sum(pos_loss) / (BATCH * (K - 1))
            + jnp.sum(neg_loss) / (BATCH * NS))

    @pl.when(i == 0)
    def _():
        out_ref[0, 0] = 0.0

    out_ref[0, 0] += part


@jax.jit
def _tc_loss(gathered):
    out = pl.pallas_call(
        _tc_loss_body,
        grid=(_NBLK,),
        in_specs=[pl.BlockSpec((NCH, _BB, D), lambda i: (0, i, 0))],
        out_specs=pl.BlockSpec(memory_space=pltpu.SMEM),
        out_shape=jax.ShapeDtypeStruct((1, 1), jnp.float32),
    )(gathered)
    return out[0, 0]


def kernel(walks, neg_samples, embedding):
    idx_all = jnp.concatenate(
        [walks.astype(jnp.int32).T, neg_samples.astype(jnp.int32).T], axis=0)
    gathered = _sc_gather(idx_all, embedding)
    return _tc_loss(gathered)


# speedup vs baseline: 2.0066x; 1.1364x over previous
import jax
import jax.numpy as jnp
from jax import lax
from jax.experimental import pallas as pl
from jax.experimental.pallas import tpu as pltpu
from jax.experimental.pallas import tpu_sc as plsc

D = 128
DC = D // 16
BATCH = 4096
K = 10
NS = 5
NCH = K + NS
NPAIR = NCH - 1
NW = 32
BPW = BATCH // NW
SUB = 16
NG = BPW // SUB


def _sc_body(idx_hbm, emb_hbm, out_hbm, idx_v, xbuf, accflat, dots_v,
             sem0, sem1):
    c = lax.axis_index("c")
    s = lax.axis_index("s")
    wid = s * 2 + c
    base = wid * BPW
    pltpu.sync_copy(idx_hbm.at[:, pl.ds(base, BPW)], idx_v)

    sems = (sem0, sem1)

    def fire(g, buf):
        handles = []
        for k in range(NCH):
            handles.append(pltpu.async_copy(
                emb_hbm.at[idx_v.at[k, pl.ds(g * SUB, SUB)]],
                xbuf.at[buf, k], sems[buf]))
        return handles

    def drain(handles):
        for h in handles:
            h.wait()

    lane = lax.iota(jnp.int32, 16)

    def compute(g, buf):
        def row_body(r, carry):
            x0c = [xbuf[buf, 0, r, pl.ds(cc * 16, 16)] for cc in range(DC)]
            for k in range(1, NCH):
                acc = x0c[0] * xbuf[buf, k, r, pl.ds(0, 16)]
                for cc in range(1, DC):
                    acc = acc + x0c[cc] * xbuf[buf, k, r, pl.ds(cc * 16, 16)]
                plsc.store_scatter(
                    accflat, [lane * 16 + ((k - 1) * 256 + r)], acc)
            return carry

        lax.fori_loop(0, SUB, row_body, 0, unroll=False)

        for k in range(NPAIR):
            dot = accflat[pl.ds(k * 256, 16)]
            for l in range(1, 16):
                dot = dot + accflat[pl.ds(k * 256 + l * 16, 16)]
            dots_v[k, pl.ds(g * SUB, SUB)] = dot

    handles = fire(0, 0)
    for g in range(NG):
        nxt = None
        if g + 1 < NG:
            nxt = fire(g + 1, (g + 1) % 2)
        drain(handles)
        compute(g, g % 2)
        handles = nxt

    pltpu.sync_copy(dots_v, out_hbm.at[:, pl.ds(base, BPW)])


@jax.jit
def _sc_dots(idx_all, embedding):
    mesh = plsc.VectorSubcoreMesh(core_axis_name="c", subcore_axis_name="s")
    return pl.kernel(
        _sc_body,
        out_type=jax.ShapeDtypeStruct((NPAIR, BATCH), jnp.float32),
        mesh=mesh,
        compiler_params=pltpu.CompilerParams(needs_layout_passes=False),
        scratch_types=[
            pltpu.VMEM((NCH, BPW), jnp.int32),
            pltpu.VMEM((2, NCH, SUB, D), jnp.float32),
            pltpu.VMEM((NPAIR * 256,), jnp.float32),
            pltpu.VMEM((NPAIR, BPW), jnp.float32),
            pltpu.SemaphoreType.DMA,
            pltpu.SemaphoreType.DMA,
        ],
    )(idx_all, embedding)


def _tc_loss_body(d_ref, out_ref):
    dots = d_ref[...]
    pos = dots[: K - 1]
    neg = dots[K - 1:]
    pos_loss = -jnp.log(jax.nn.sigmoid(pos) + 1e-08)
    neg_loss = -jnp.log(1.0 - jax.nn.sigmoid(neg) + 1e-08)
    out_ref[0, 0] = (jnp.sum(pos_loss) / (BATCH * (K - 1))
                     + jnp.sum(neg_loss) / (BATCH * NS))


@jax.jit
def _tc_loss(dots):
    out = pl.pallas_call(
        _tc_loss_body,
        out_specs=pl.BlockSpec(memory_space=pltpu.SMEM),
        out_shape=jax.ShapeDtypeStruct((1, 1), jnp.float32),
    )(dots)
    return out[0, 0]


def kernel(walks, neg_samples, embedding):
    idx_all = jnp.concatenate(
        [walks.astype(jnp.int32).T, neg_samples.astype(jnp.int32).T], axis=0)
    dots = _sc_dots(idx_all, embedding)
    return _tc_loss(dots)


# speedup vs baseline: 2.1338x; 1.0634x over previous
import jax
import jax.numpy as jnp
from jax import lax
from jax.experimental import pallas as pl
from jax.experimental.pallas import tpu as pltpu
from jax.experimental.pallas import tpu_sc as plsc

D = 128
DC = D // 16
BATCH = 4096
K = 10
NS = 5
NCH = K + NS
NPAIR = NCH - 1
NW = 32
BPW = BATCH // NW
SUB = 16
NG = BPW // SUB


def _tree_sum(vs):
    vs = list(vs)
    while len(vs) > 1:
        nxt = [vs[i] + vs[i + 1] for i in range(0, len(vs) - 1, 2)]
        if len(vs) % 2:
            nxt.append(vs[-1])
        vs = nxt
    return vs[0]


def _sc_body(walks_hbm, neg_hbm, emb_hbm, out_hbm,
             walks_v, neg_v, idx_v, xbuf, accflat, dots_v, sem0, sem1):
    c = lax.axis_index("c")
    s = lax.axis_index("s")
    wid = s * 2 + c
    base = wid * BPW
    pltpu.sync_copy(walks_hbm.at[pl.ds(base * K, BPW * K)], walks_v)
    pltpu.sync_copy(neg_hbm.at[pl.ds(base * NS, BPW * NS)], neg_v)

    lane = lax.iota(jnp.int32, 16)

    for g in range(NG):
        rows = lane + g * SUB
        for k in range(K):
            vec = plsc.load_gather(walks_v, [rows * K + k])
            idx_v[k, pl.ds(g * SUB, SUB)] = vec
        for k in range(NS):
            vec = plsc.load_gather(neg_v, [rows * NS + k])
            idx_v[K + k, pl.ds(g * SUB, SUB)] = vec

    sems = (sem0, sem1)

    def fire(g, buf):
        handles = []
        for k in range(NCH):
            handles.append(pltpu.async_copy(
                emb_hbm.at[idx_v.at[k, pl.ds(g * SUB, SUB)]],
                xbuf.at[buf, k], sems[buf]))
        return handles

    def compute(g, buf):
        def row_body(r, carry):
            x0c = [xbuf[buf, 0, r, pl.ds(cc * 16, 16)] for cc in range(DC)]
            for k in range(1, NCH):
                acc = _tree_sum(
                    [x0c[cc] * xbuf[buf, k, r, pl.ds(cc * 16, 16)]
                     for cc in range(DC)])
                plsc.store_scatter(
                    accflat, [lane * 16 + ((k - 1) * 256 + r)], acc)
            return carry

        lax.fori_loop(0, SUB, row_body, 0, unroll=False)

        for k in range(NPAIR):
            dot = _tree_sum(
                [accflat[pl.ds(k * 256 + l * 16, 16)] for l in range(16)])
            dots_v[k, pl.ds(g * SUB, SUB)] = dot

    handles = fire(0, 0)
    for g in range(NG):
        nxt = None
        if g + 1 < NG:
            nxt = fire(g + 1, (g + 1) % 2)
        for h in handles:
            h.wait()
        compute(g, g % 2)
        handles = nxt

    pltpu.sync_copy(dots_v, out_hbm.at[:, pl.ds(base, BPW)])


@jax.jit
def _sc_dots(walks_flat, neg_flat, embedding):
    mesh = plsc.VectorSubcoreMesh(core_axis_name="c", subcore_axis_name="s")
    return pl.kernel(
        _sc_body,
        out_type=jax.ShapeDtypeStruct((NPAIR, BATCH), jnp.float32),
        mesh=mesh,
        compiler_params=pltpu.CompilerParams(needs_layout_passes=False),
        scratch_types=[
            pltpu.VMEM((BPW * K,), jnp.int32),
            pltpu.VMEM((BPW * NS,), jnp.int32),
            pltpu.VMEM((NCH, BPW), jnp.int32),
            pltpu.VMEM((2, NCH, SUB, D), jnp.float32),
            pltpu.VMEM((NPAIR * 256,), jnp.float32),
            pltpu.VMEM((NPAIR, BPW), jnp.float32),
            pltpu.SemaphoreType.DMA,
            pltpu.SemaphoreType.DMA,
        ],
    )(walks_flat, neg_flat, embedding)


def _tc_loss_body(d_ref, out_ref):
    dots = d_ref[...]
    pos = dots[: K - 1]
    neg = dots[K - 1:]
    pos_loss = -jnp.log(jax.nn.sigmoid(pos) + 1e-08)
    neg_loss = -jnp.log(1.0 - jax.nn.sigmoid(neg) + 1e-08)
    out_ref[0, 0] = (jnp.sum(pos_loss) / (BATCH * (K - 1))
                     + jnp.sum(neg_loss) / (BATCH * NS))


@jax.jit
def _tc_loss(dots):
    out = pl.pallas_call(
        _tc_loss_body,
        out_specs=pl.BlockSpec(memory_space=pltpu.SMEM),
        out_shape=jax.ShapeDtypeStruct((1, 1), jnp.float32),
    )(dots)
    return out[0, 0]


def kernel(walks, neg_samples, embedding):
    walks_flat = walks.astype(jnp.int32).reshape(-1)
    neg_flat = neg_samples.astype(jnp.int32).reshape(-1)
    dots = _sc_dots(walks_flat, neg_flat, embedding)
    return _tc_loss(dots)
